# baseline (speedup 1.0000x reference)
import functools
import jax
import jax.numpy as jnp
from jax import lax
from jax.experimental import pallas as pl
from jax.experimental.pallas import tpu as pltpu, tpu_sc as plsc

_NBUF = 2
_CHUNK = 504


def _sc_copy_add(n, d):
    info = plsc.get_sparse_core_info()
    NC, NS = info.num_cores, info.num_subcores
    NW = NC * NS
    nch = n // _CHUNK
    tail = n - nch * _CHUNK
    chunks_w, rem = divmod(nch, NW)
    ngroups, grem = divmod(chunks_w, _NBUF)
    assert ngroups >= 2 and rem == 0
    mesh = plsc.VectorSubcoreMesh(core_axis_name="c", subcore_axis_name="s")

    @functools.partial(
        pl.kernel,
        mesh=mesh,
        out_type=jax.ShapeDtypeStruct((n, d), jnp.float32),
        scratch_types=(
            [pltpu.VMEM((_CHUNK, d), jnp.float32) for _ in range(_NBUF)]
            + [pltpu.VMEM((1, d), jnp.float32)]
            + [pltpu.SemaphoreType.DMA for _ in range(2 * _NBUF)]
        ),
    )
    def k(x_hbm, v_hbm, out_hbm, *refs):
        bufs = refs[:_NBUF]
        val_v = refs[_NBUF]
        sin = refs[_NBUF + 1 : _NBUF + 1 + _NBUF]
        sout = refs[_NBUF + 1 + _NBUF :]

        wid = lax.axis_index("s") * NC + lax.axis_index("c")

        def _off(i):
            return (i * NW + wid) * _CHUNK

        def in_cp(c, b):
            return pltpu.make_async_copy(
                x_hbm.at[pl.ds(_off(c), _CHUNK)], bufs[b], sin[b]
            )

        def out_cp(c, b):
            return pltpu.make_async_copy(
                bufs[b], out_hbm.at[pl.ds(_off(c), _CHUNK)], sout[b]
            )

        for b in range(_NBUF):
            in_cp(b, b).start()
        for b in range(_NBUF):
            in_cp(b, b).wait()
            if b == 0:
                @pl.when(wid == 0)
                def _():
                    pltpu.sync_copy(v_hbm, val_v)
                    for j in range(d // 16):
                        sl = pl.ds(j * 16, 16)
                        bufs[0][0, sl] = bufs[0][0, sl] + val_v[0, sl]
            out_cp(b, b).start()

        @pl.loop(1, ngroups)
        def _(g):
            c0 = g * _NBUF
            for b in range(_NBUF):
                out_cp(c0 - _NBUF + b, b).wait()
                in_cp(c0 + b, b).start()
            for b in range(_NBUF):
                in_cp(c0 + b, b).wait()
                out_cp(c0 + b, b).start()

        for b in range(_NBUF):
            out_cp((ngroups - 1) * _NBUF + b, b).wait()

        for e in range(grem):
            ce = ngroups * _NBUF + e
            pltpu.sync_copy(x_hbm.at[pl.ds(_off(ce), _CHUNK)], bufs[0])
            pltpu.sync_copy(bufs[0], out_hbm.at[pl.ds(_off(ce), _CHUNK)])


        if tail:
            @pl.when(wid == NW - 1)
            def _():
                pltpu.sync_copy(x_hbm.at[pl.ds(nch * _CHUNK, tail)], bufs[0].at[pl.ds(0, tail)])
                pltpu.sync_copy(bufs[0].at[pl.ds(0, tail)], out_hbm.at[pl.ds(nch * _CHUNK, tail)])

    return k


def kernel(main_tensor, value):
    n, d = main_tensor.shape
    x = main_tensor.reshape(n // 2, 2 * d)
    v = value.reshape(1, 2 * d)
    out = _sc_copy_add(n // 2, 2 * d)(x, v)
    return out.reshape(n, d)


# speedup vs baseline: 6.5522x; 6.5522x over previous
import functools
import jax
import jax.numpy as jnp
from jax import lax
from jax.experimental import pallas as pl
from jax.experimental.pallas import tpu as pltpu, tpu_sc as plsc

_NBUF = 2
_RG = 8
_CB = 8064


def _sc_copy_add_t(d, n):
    info = plsc.get_sparse_core_info()
    NC, NS = info.num_cores, info.num_subcores
    NW = NC * NS
    nrg = d // _RG
    ncg = NW // nrg
    cols_main = (n // 128 // ncg) * 128 * ncg
    cols_g = cols_main // ncg
    edge = n - cols_main
    nch, chrem = divmod(cols_g, _CB)
    ngroups, grem = divmod(nch, _NBUF)
    assert chrem == 0 and ngroups >= 2
    mesh = plsc.VectorSubcoreMesh(core_axis_name="c", subcore_axis_name="s")

    @functools.partial(
        pl.kernel,
        mesh=mesh,
        out_type=jax.ShapeDtypeStruct((d, n), jnp.float32),
        scratch_types=(
            [pltpu.VMEM((_RG, _CB), jnp.float32) for _ in range(_NBUF)]
            + [pltpu.VMEM((_RG, 16), jnp.float32)]
            + [pltpu.SemaphoreType.DMA for _ in range(2 * _NBUF)]
        ),
    )
    def k(x_hbm, vp_hbm, out_hbm, *refs):
        bufs = refs[:_NBUF]
        val_v = refs[_NBUF]
        sin = refs[_NBUF + 1 : _NBUF + 1 + _NBUF]
        sout = refs[_NBUF + 1 + _NBUF :]

        wid = lax.axis_index("s") * NC + lax.axis_index("c")
        rg = wid // ncg
        cg = lax.rem(wid, ncg)
        r0 = rg * _RG
        c0 = cg * cols_g

        def in_cp(c, b):
            return pltpu.make_async_copy(
                x_hbm.at[pl.ds(r0, _RG), pl.ds(c0 + c * _CB, _CB)],
                bufs[b],
                sin[b],
            )

        def out_cp(c, b):
            return pltpu.make_async_copy(
                bufs[b],
                out_hbm.at[pl.ds(r0, _RG), pl.ds(c0 + c * _CB, _CB)],
                sout[b],
            )

        for b in range(_NBUF):
            in_cp(b, b).start()
        for b in range(_NBUF):
            in_cp(b, b).wait()
            if b == 0:
                @pl.when(cg == 0)
                def _():
                    pltpu.sync_copy(vp_hbm.at[pl.ds(r0, _RG)], val_v)
                    for r in range(_RG):
                        sl = pl.ds(0, 16)
                        bufs[0][r, sl] = bufs[0][r, sl] + val_v[r, sl]
            out_cp(b, b).start()

        @pl.loop(1, ngroups)
        def _(g):
            cbase = g * _NBUF
            for b in range(_NBUF):
                out_cp(cbase - _NBUF + b, b).wait()
                in_cp(cbase + b, b).start()
            for b in range(_NBUF):
                in_cp(cbase + b, b).wait()
                out_cp(cbase + b, b).start()

        for b in range(_NBUF):
            out_cp((ngroups - 1) * _NBUF + b, b).wait()

        for e in range(grem):
            ce = ngroups * _NBUF + e
            pltpu.sync_copy(
                x_hbm.at[pl.ds(r0, _RG), pl.ds(c0 + ce * _CB, _CB)], bufs[0]
            )
            pltpu.sync_copy(
                bufs[0], out_hbm.at[pl.ds(r0, _RG), pl.ds(c0 + ce * _CB, _CB)]
            )

    return k


def kernel(main_tensor, value):
    n, d = main_tensor.shape
    xt = main_tensor.T
    vpad = jnp.zeros((d, 16), dtype=value.dtype).at[:, : value.shape[0]].set(value.T)
    out = _sc_copy_add_t(d, n)(xt, vpad).T
    cols_main = (n // 128 // 4) * 128 * 4
    if cols_main < n:
        tail_rows = jax.lax.slice(main_tensor, (cols_main, 0), (n, d))
        out = jax.lax.dynamic_update_slice(out, tail_rows, (cols_main, 0))
    return out


# speedup vs baseline: 6.8484x; 1.0452x over previous
import functools
import jax
import jax.numpy as jnp
from jax import lax
from jax.experimental import pallas as pl
from jax.experimental.pallas import tpu as pltpu, tpu_sc as plsc

_NBUF = 3
_RG = 8
_CB = 3968


def _sc_copy_add_t(d, n):
    info = plsc.get_sparse_core_info()
    NC, NS = info.num_cores, info.num_subcores
    NW = NC * NS
    nrg = d // _RG
    ncg = NW // nrg
    cols_main = (n // 128 // ncg) * 128 * ncg
    cols_g = cols_main // ncg
    edge = n - cols_main
    nch, chrem = divmod(cols_g, _CB)
    ngroups, grem = divmod(nch, _NBUF)
    assert chrem == 0 and ngroups >= 2
    mesh = plsc.VectorSubcoreMesh(core_axis_name="c", subcore_axis_name="s")

    @functools.partial(
        pl.kernel,
        mesh=mesh,
        out_type=jax.ShapeDtypeStruct((d, n), jnp.float32),
        scratch_types=(
            [pltpu.VMEM((_RG, _CB), jnp.float32) for _ in range(_NBUF)]
            + [pltpu.VMEM((_RG, 16), jnp.float32)]
            + [pltpu.SemaphoreType.DMA for _ in range(2 * _NBUF)]
        ),
    )
    def k(x_hbm, vp_hbm, out_hbm, *refs):
        bufs = refs[:_NBUF]
        val_v = refs[_NBUF]
        sin = refs[_NBUF + 1 : _NBUF + 1 + _NBUF]
        sout = refs[_NBUF + 1 + _NBUF :]

        wid = lax.axis_index("s") * NC + lax.axis_index("c")
        rg = wid // ncg
        cg = lax.rem(wid, ncg)
        r0 = rg * _RG
        c0 = cg * cols_g

        def in_cp(c, b):
            return pltpu.make_async_copy(
                x_hbm.at[pl.ds(r0, _RG), pl.ds(c0 + c * _CB, _CB)],
                bufs[b],
                sin[b],
            )

        def out_cp(c, b):
            return pltpu.make_async_copy(
                bufs[b],
                out_hbm.at[pl.ds(r0, _RG), pl.ds(c0 + c * _CB, _CB)],
                sout[b],
            )

        for b in range(_NBUF):
            in_cp(b, b).start()
        for b in range(_NBUF):
            in_cp(b, b).wait()
            if b == 0:
                @pl.when(cg == 0)
                def _():
                    pltpu.sync_copy(vp_hbm.at[pl.ds(r0, _RG)], val_v)
                    for r in range(_RG):
                        sl = pl.ds(0, 16)
                        bufs[0][r, sl] = bufs[0][r, sl] + val_v[r, sl]
            out_cp(b, b).start()

        @pl.loop(1, ngroups)
        def _(g):
            cbase = g * _NBUF
            for b in range(_NBUF):
                out_cp(cbase - _NBUF + b, b).wait()
                in_cp(cbase + b, b).start()
            for b in range(_NBUF):
                in_cp(cbase + b, b).wait()
                out_cp(cbase + b, b).start()

        for b in range(_NBUF):
            out_cp((ngroups - 1) * _NBUF + b, b).wait()

        for e in range(grem):
            ce = ngroups * _NBUF + e
            pltpu.sync_copy(
                x_hbm.at[pl.ds(r0, _RG), pl.ds(c0 + ce * _CB, _CB)], bufs[0]
            )
            pltpu.sync_copy(
                bufs[0], out_hbm.at[pl.ds(r0, _RG), pl.ds(c0 + ce * _CB, _CB)]
            )

    return k


def kernel(main_tensor, value):
    n, d = main_tensor.shape
    xt = main_tensor.T
    vpad = jnp.zeros((d, 16), dtype=value.dtype).at[:, : value.shape[0]].set(value.T)
    out = _sc_copy_add_t(d, n)(xt, vpad).T
    cols_main = (n // 128 // 4) * 128 * 4
    if cols_main < n:
        tail_rows = jax.lax.slice(main_tensor, (cols_main, 0), (n, d))
        out = jax.lax.dynamic_update_slice(out, tail_rows, (cols_main, 0))
    return out


# speedup vs baseline: 6.9508x; 1.0150x over previous
import functools
import jax
import jax.numpy as jnp
from jax import lax
from jax.experimental import pallas as pl
from jax.experimental.pallas import tpu as pltpu, tpu_sc as plsc

_NBUF = 3
_RG = 8
_CB = 2688


def _sc_copy_add_t(d, n):
    info = plsc.get_sparse_core_info()
    NC, NS = info.num_cores, info.num_subcores
    NW = NC * NS
    nrg = d // _RG
    ncg = NW // nrg
    cols_main = (n // 128 // ncg) * 128 * ncg
    cols_g = cols_main // ncg
    edge = n - cols_main
    nch, chrem = divmod(cols_g, _CB)
    ngroups, grem = divmod(nch, _NBUF)
    assert chrem == 0 and ngroups >= 2
    mesh = plsc.VectorSubcoreMesh(core_axis_name="c", subcore_axis_name="s")

    @functools.partial(
        pl.kernel,
        mesh=mesh,
        out_type=jax.ShapeDtypeStruct((d, n), jnp.float32),
        scratch_types=(
            [pltpu.VMEM((_RG, _CB), jnp.float32) for _ in range(_NBUF)]
            + [pltpu.VMEM((_RG, 16), jnp.float32)]
            + [pltpu.SemaphoreType.DMA for _ in range(2 * _NBUF)]
        ),
    )
    def k(x_hbm, vp_hbm, out_hbm, *refs):
        bufs = refs[:_NBUF]
        val_v = refs[_NBUF]
        sin = refs[_NBUF + 1 : _NBUF + 1 + _NBUF]
        sout = refs[_NBUF + 1 + _NBUF :]

        wid = lax.axis_index("s") * NC + lax.axis_index("c")
        rg = wid // ncg
        cg = lax.rem(wid, ncg)
        r0 = rg * _RG
        c0 = cg * cols_g

        def in_cp(c, b):
            return pltpu.make_async_copy(
                x_hbm.at[pl.ds(r0, _RG), pl.ds(c0 + c * _CB, _CB)],
                bufs[b],
                sin[b],
            )

        def out_cp(c, b):
            return pltpu.make_async_copy(
                bufs[b],
                out_hbm.at[pl.ds(r0, _RG), pl.ds(c0 + c * _CB, _CB)],
                sout[b],
            )

        for b in range(_NBUF):
            in_cp(b, b).start()
        for b in range(_NBUF):
            in_cp(b, b).wait()
            if b == 0:
                @pl.when(cg == 0)
                def _():
                    pltpu.sync_copy(vp_hbm.at[pl.ds(r0, _RG)], val_v)
                    for r in range(_RG):
                        sl = pl.ds(0, 16)
                        bufs[0][r, sl] = bufs[0][r, sl] + val_v[r, sl]
            out_cp(b, b).start()

        @pl.loop(1, ngroups)
        def _(g):
            cbase = g * _NBUF
            for b in range(_NBUF):
                out_cp(cbase - _NBUF + b, b).wait()
                in_cp(cbase + b, b).start()
            for b in range(_NBUF):
                in_cp(cbase + b, b).wait()
                out_cp(cbase + b, b).start()

        for b in range(_NBUF):
            out_cp((ngroups - 1) * _NBUF + b, b).wait()

        for e in range(grem):
            ce = ngroups * _NBUF + e
            pltpu.sync_copy(
                x_hbm.at[pl.ds(r0, _RG), pl.ds(c0 + ce * _CB, _CB)], bufs[0]
            )
            pltpu.sync_copy(
                bufs[0], out_hbm.at[pl.ds(r0, _RG), pl.ds(c0 + ce * _CB, _CB)]
            )

    return k


def kernel(main_tensor, value):
    n, d = main_tensor.shape
    xt = main_tensor.T
    vpad = jnp.zeros((d, 16), dtype=value.dtype).at[:, : value.shape[0]].set(value.T)
    out = _sc_copy_add_t(d, n)(xt, vpad).T
    cols_main = (n // 128 // 4) * 128 * 4
    if cols_main < n:
        tail_rows = jax.lax.slice(main_tensor, (cols_main, 0), (n, d))
        out = jax.lax.dynamic_update_slice(out, tail_rows, (cols_main, 0))
    return out
